# baseline (speedup 1.0000x reference)
import functools

import jax
import jax.numpy as jnp
from jax import lax
from jax.experimental import pallas as pl
from jax.experimental.pallas import tpu as pltpu
from jax.experimental.pallas import tpu_sc as plsc

_NUM_CORES = 2
_NUM_SUBCORES = 16
_LANES = 16
_DBLK = 256


@functools.partial(jax.jit, static_argnums=(1,))
def _one_hot_sc(x, d):
    j, b = x.shape
    nw = _NUM_CORES * _NUM_SUBCORES
    bw = b // nw
    assert b % nw == 0 and bw == 128
    nchunk = bw // _LANES
    dblocks = []
    d0 = 0
    while d0 < d:
        dblocks.append((d0, min(_DBLK, d - d0)))
        d0 += _DBLK
    nq = len(dblocks)
    assert nq % 2 == 0

    mesh = plsc.VectorSubcoreMesh(
        core_axis_name="c", subcore_axis_name="s",
        num_cores=_NUM_CORES, num_subcores=_NUM_SUBCORES)

    @functools.partial(
        pl.kernel,
        mesh=mesh,
        compiler_params=pltpu.CompilerParams(needs_layout_passes=False),
        out_type=jax.ShapeDtypeStruct((j, d, b), jnp.float32),
        scratch_types=[
            pltpu.VMEM((j, bw), jnp.int32),
            pltpu.VMEM((_DBLK, bw), jnp.float32),
            pltpu.VMEM((_DBLK, bw), jnp.float32),
            pltpu.SemaphoreType.DMA,
            pltpu.SemaphoreType.DMA,
        ],
    )
    def run(x_hbm, out_hbm, xv, slab0, slab1, sem0, sem1):
        cid = lax.axis_index("c")
        sid = lax.axis_index("s")
        wid = sid * _NUM_CORES + cid
        b0 = wid * bw

        pltpu.sync_copy(x_hbm.at[pl.ds(0, j), pl.ds(b0, bw)], xv)

        zf = jnp.zeros((_LANES,), jnp.float32)
        onef = jnp.full((_LANES,), 1.0, jnp.float32)
        lane = lax.iota(jnp.int32, _LANES)

        def zero_body(i, carry):
            r = i // (bw // _LANES)
            c = (i % (bw // _LANES)) * _LANES
            slab0[r, pl.ds(c, _LANES)] = zf
            slab1[r, pl.ds(c, _LANES)] = zf
            return carry

        lax.fori_loop(0, _DBLK * bw // _LANES, zero_body, 0)

        slabs = (slab0, slab1)
        sems = (sem0, sem1)

        def xcol(jj):
            cols = jnp.full((_LANES,), jj, jnp.int32)
            return [plsc.load_gather(xv, [cols, lane + c * _LANES])
                    for c in range(nchunk)]

        def scatter(slab, xs, dlo, dsz, val):
            for c in range(nchunk):
                xc = xs[c]
                row = xc - dlo
                mask = (xc >= dlo) & (xc < dlo + dsz)
                plsc.store_scatter(slab, [row, lane + c * _LANES],
                                   val, mask=mask)

        def dma(slab, sem, jj, dlo, dsz):
            return pltpu.async_copy(
                slab.at[pl.ds(0, dsz)],
                out_hbm.at[jj, pl.ds(dlo, dsz), pl.ds(b0, bw)], sem)

        def drain(slab, sem, jj, dlo, dsz):
            pltpu.make_async_copy(
                slab.at[pl.ds(0, dsz)],
                out_hbm.at[jj, pl.ds(dlo, dsz), pl.ds(b0, bw)], sem).wait()

        def unit(jj, q, first):
            dlo, dsz = dblocks[q]
            slab, sem = slabs[q % 2], sems[q % 2]
            qp = (q + nq - 2) % nq
            dlop, dszp = dblocks[qp]
            jjp = jj - (1 if q < 2 else 0)
            if not first:
                drain(slab, sem, jjp, dlop, dszp)
                scatter(slab, xcol(jjp), dlop, dszp, zf)
            xs = xcol(jj)
            scatter(slab, xs, dlo, dsz, onef)
            dma(slab, sem, jj, dlo, dsz)

        unit(0, 0, True)
        unit(0, 1, True)

        def jj_body(jj, carry):
            for q in range(2, nq):
                unit(jj, q, False)
            for q in range(2):
                unit(jj + 1, q, False)
            return carry

        lax.fori_loop(0, j - 1, jj_body, 0)
        for q in range(2, nq):
            unit(j - 1, q, False)

        for q in (nq - 2, nq - 1):
            dlo, dsz = dblocks[q]
            drain(slabs[q % 2], sems[q % 2], j - 1, dlo, dsz)

    return run(x)


def kernel(X_in, ones):
    d = ones.shape[0]
    out = _one_hot_sc(X_in.T, d)
    return jnp.transpose(out, (2, 1, 0))
